# baseline (speedup 1.0000x reference)
import jax
import jax.numpy as jnp
from jax.experimental import pallas as pl
from jax.experimental.pallas import tpu as pltpu

RANK = 6
QUAD = 4


def _spatial_factors(xy_lamb):
    d = jnp.arange(-64, 65, dtype=jnp.float32)
    r2 = d[:, None] ** 2 + d[None, :] ** 2
    K = jnp.exp(-jnp.sqrt(r2) / xy_lamb)
    t = jnp.arange(RANK, dtype=jnp.float32) / (RANK - 1)
    sig = jnp.clip(xy_lamb * 0.15 * (55.0 / 1.5) ** t, 0.7, 70.0)
    G = jnp.exp(-(d[:, None] ** 2) / (2.0 * sig[None, :] ** 2))
    hi = jax.lax.Precision.HIGHEST
    KG = jnp.matmul(K, G, precision=hi)
    M = jnp.matmul(G.T, KG, precision=hi)
    M = M + (1e-5 / RANK) * jnp.trace(M) * jnp.eye(RANK, dtype=jnp.float32)
    L = [[None] * RANK for _ in range(RANK)]
    for i in range(RANK):
        for j in range(i + 1):
            s = M[i, j]
            for p in range(j):
                s = s - L[i][p] * L[j][p]
            if i == j:
                L[i][j] = jnp.sqrt(s)
            else:
                L[i][j] = s / L[j][j]
    Linv = [[None] * RANK for _ in range(RANK)]
    zero = jnp.zeros((), jnp.float32)
    for j in range(RANK):
        Linv[j][j] = 1.0 / L[j][j]
        for i in range(j + 1, RANK):
            s = zero
            for p in range(j, i):
                s = s + L[i][p] * Linv[p][j]
            Linv[i][j] = -s / L[i][i]
        for i in range(j):
            Linv[i][j] = zero
    LinvM = jnp.stack([jnp.stack(row) for row in Linv])
    return jnp.matmul(LinvM, KG.T, precision=hi)


def _toeplitz_all(Z):
    zp = jnp.pad(Z, ((0, 0), (64, 64)))
    w = zp[:, 1:256]
    big = jnp.pad(w, ((0, 0), (0, 1)))
    rep = jnp.tile(big, (1, 129))[:, : 128 * 257].reshape(-1, 128, 257)
    hank = rep[:, :, :128]
    return hank[:, :, ::-1]


def _feature_matrix(C, lamb):
    nb = C + 1
    c = jnp.arange(C)
    m = (c[:, None] - c[None, :]) % nb
    dist = jnp.where(m <= nb // 2, m, m - nb)
    return jnp.exp(-jnp.abs(dist).astype(jnp.float32) / lamb)


def _body(x_ref, g_ref, t_ref, f_ref, scal_ref, o_ref, u_scr, s_scr):
    C, H, W = x_ref.shape[1], x_ref.shape[2], x_ref.shape[3]
    xb = x_ref[0]
    y = xb * xb
    yf = y.reshape(C * H, W)
    u_scr[...] = jnp.dot(
        yf.astype(jnp.bfloat16), g_ref[...], preferred_element_type=jnp.float32
    ).astype(jnp.bfloat16)
    tc = t_ref[...]
    for c0 in range(0, C, QUAD):
        cols = []
        for c in range(c0, c0 + QUAD):
            u_c = u_scr[c * H:(c + 1) * H, :]
            cols.append(
                jnp.concatenate(
                    [u_c[:, kr * 128:(kr + 1) * 128] for kr in range(RANK)],
                    axis=0,
                )
            )
        ucat = jnp.concatenate(cols, axis=1)
        sp = jnp.dot(tc, ucat, preferred_element_type=jnp.float32).astype(
            jnp.bfloat16
        )
        for q in range(QUAD):
            c = c0 + q
            s_scr[c * H:(c + 1) * H, :] = sp[:, q * 128:(q + 1) * 128]
    s3 = s_scr[...].reshape(C, H, W)
    t3 = jnp.einsum(
        "dc,chw->dhw", f_ref[...], s3, preferred_element_type=jnp.float32
    )
    be = scal_ref[0]
    kk = scal_ref[1]
    den = jnp.exp2(be * jnp.log2(t3 + kk))
    o_ref[0] = y / (den + 1e-6)


def kernel(x, lamb, xy_lamb, alpha, beta, k):
    B, C, H, W = x.shape
    lam = lamb[0]
    xyl = xy_lamb[0]
    Z = _spatial_factors(xyl)
    Tall = _toeplitz_all(Z)
    tcat = jnp.concatenate([Tall[i] for i in range(RANK)], axis=1)
    gcat = jnp.concatenate([Tall[i].T for i in range(RANK)], axis=1)
    norm = (lam + 1e-6) * (xyl + 1e-6) ** 2
    fm = (_feature_matrix(C, lam) * (alpha[0] / norm)).astype(jnp.bfloat16)
    scal = jnp.stack([beta[0], k[0]])

    return pl.pallas_call(
        _body,
        grid=(B,),
        in_specs=[
            pl.BlockSpec((1, C, H, W), lambda b: (b, 0, 0, 0)),
            pl.BlockSpec((W, 128 * RANK), lambda b: (0, 0)),
            pl.BlockSpec((H, 128 * RANK), lambda b: (0, 0)),
            pl.BlockSpec((C, C), lambda b: (0, 0)),
            pl.BlockSpec(memory_space=pltpu.SMEM),
        ],
        out_specs=pl.BlockSpec((1, C, H, W), lambda b: (b, 0, 0, 0)),
        out_shape=jax.ShapeDtypeStruct((B, C, H, W), jnp.float32),
        scratch_shapes=[
            pltpu.VMEM((C * H, 128 * RANK), jnp.bfloat16),
            pltpu.VMEM((C * H, W), jnp.bfloat16),
        ],
        compiler_params=pltpu.CompilerParams(
            dimension_semantics=("parallel",),
            vmem_limit_bytes=100 * 1024 * 1024,
        ),
    )(x, gcat.astype(jnp.bfloat16), tcat.astype(jnp.bfloat16), fm, scal)


# speedup vs baseline: 1.3219x; 1.3219x over previous
import jax
import jax.numpy as jnp
from jax.experimental import pallas as pl
from jax.experimental.pallas import tpu as pltpu

RANK = 5
QUAD = 4
TAUS = (0.3, 0.6817316198804996, 1.5491933384829666, 3.5204469471735735, 8.0)
CS = (
    0.22476744850479666,
    0.36820675113371154,
    0.27861523349776374,
    0.01354796403485517,
    -0.0006253867445723815,
)
DELTA = 1.0 - sum(CS)


def _toeplitz_all(Z):
    zp = jnp.pad(Z, ((0, 0), (64, 64)))
    w = zp[:, 1:256]
    big = jnp.pad(w, ((0, 0), (0, 1)))
    rep = jnp.tile(big, (1, 129))[:, : 128 * 257].reshape(-1, 128, 257)
    hank = rep[:, :, :128]
    return hank[:, :, ::-1]


def _feature_matrix(C, lamb):
    nb = C + 1
    c = jnp.arange(C)
    m = (c[:, None] - c[None, :]) % nb
    dist = jnp.where(m <= nb // 2, m, m - nb)
    return jnp.exp(-jnp.abs(dist).astype(jnp.float32) / lamb)


def _body(x_ref, g_ref, t_ref, f_ref, scal_ref, o_ref, u_scr, s_scr):
    C, H, W = x_ref.shape[1], x_ref.shape[2], x_ref.shape[3]
    xb = x_ref[0]
    y = xb * xb
    yf = y.astype(jnp.bfloat16).reshape(C * H, W)
    u_scr[...] = jnp.dot(
        yf, g_ref[...], preferred_element_type=jnp.float32
    ).astype(jnp.bfloat16)
    tc = t_ref[...]
    for c0 in range(0, C, QUAD):
        cols = []
        for c in range(c0, c0 + QUAD):
            u_c = u_scr[c * H:(c + 1) * H, :]
            cols.append(
                jnp.concatenate(
                    [u_c[:, kr * 128:(kr + 1) * 128] for kr in range(RANK)],
                    axis=0,
                )
            )
        ucat = jnp.concatenate(cols, axis=1)
        sp = jnp.dot(tc, ucat, preferred_element_type=jnp.float32).astype(
            jnp.bfloat16
        )
        for q in range(QUAD):
            c = c0 + q
            s_scr[c * H:(c + 1) * H, :] = sp[:, q * 128:(q + 1) * 128]
    s3 = s_scr[...].reshape(C, H, W) + jnp.bfloat16(DELTA) * yf.reshape(C, H, W)
    t3 = jnp.einsum(
        "dc,chw->dhw", f_ref[...], s3, preferred_element_type=jnp.float32
    )
    be = scal_ref[0]
    kk = scal_ref[1]
    den = jnp.exp2(be * jnp.log2(t3 + kk))
    o_ref[0] = y / (den + 1e-6)


def kernel(x, lamb, xy_lamb, alpha, beta, k):
    B, C, H, W = x.shape
    lam = lamb[0]
    xyl = xy_lamb[0]
    d = jnp.arange(-64.0, 65.0, dtype=jnp.float32)
    tau = jnp.asarray(TAUS, jnp.float32) * xyl
    G = jnp.exp(-(d[None, :] ** 2) / (2.0 * tau[:, None] ** 2))
    gall = _toeplitz_all(G)
    tall = jnp.asarray(CS, jnp.float32)[:, None, None] * gall
    tcat = jnp.moveaxis(tall, 0, 1).reshape(H, 128 * RANK)
    gcat = jnp.moveaxis(gall, 0, 1).reshape(W, 128 * RANK)
    norm = (lam + 1e-6) * (xyl + 1e-6) ** 2
    fm = (_feature_matrix(C, lam) * (alpha[0] / norm)).astype(jnp.bfloat16)
    scal = jnp.stack([beta[0], k[0]])

    return pl.pallas_call(
        _body,
        grid=(B,),
        in_specs=[
            pl.BlockSpec((1, C, H, W), lambda b: (b, 0, 0, 0)),
            pl.BlockSpec((W, 128 * RANK), lambda b: (0, 0)),
            pl.BlockSpec((H, 128 * RANK), lambda b: (0, 0)),
            pl.BlockSpec((C, C), lambda b: (0, 0)),
            pl.BlockSpec(memory_space=pltpu.SMEM),
        ],
        out_specs=pl.BlockSpec((1, C, H, W), lambda b: (b, 0, 0, 0)),
        out_shape=jax.ShapeDtypeStruct((B, C, H, W), jnp.float32),
        scratch_shapes=[
            pltpu.VMEM((C * H, 128 * RANK), jnp.bfloat16),
            pltpu.VMEM((C * H, W), jnp.bfloat16),
        ],
        compiler_params=pltpu.CompilerParams(
            dimension_semantics=("parallel",),
            vmem_limit_bytes=100 * 1024 * 1024,
        ),
    )(x, gcat.astype(jnp.bfloat16), tcat.astype(jnp.bfloat16), fm, scal)
